# baseline (speedup 1.0000x reference)
import functools

import jax
import jax.numpy as jnp
from jax import lax
from jax.experimental import pallas as pl
from jax.experimental.pallas import tpu as pltpu
from jax.experimental.pallas import tpu_sc as plsc

_TOKENS = 32768
_HIDDEN = 768
_EXPERTS = 64
_K = 8

_NC = 2
_NS = 16
_NW = _NC * _NS
_ROWS_PER_W = _TOKENS // _NW

_MM_BLOCK = 2048


def _matmul_body(x_ref, w_ref, b_ref, out_ref):
    x = x_ref[...]
    w = w_ref[...]
    logits = lax.dot_general(
        x, w, (((1,), (1,)), ((), ())), preferred_element_type=jnp.float32
    )
    out_ref[...] = logits + b_ref[...]


def _router_logits(x, w, b):
    grid = (_TOKENS // _MM_BLOCK,)
    return pl.pallas_call(
        _matmul_body,
        grid=grid,
        in_specs=[
            pl.BlockSpec((_MM_BLOCK, _HIDDEN), lambda i: (i, 0)),
            pl.BlockSpec((_EXPERTS, _HIDDEN), lambda i: (0, 0)),
            pl.BlockSpec((1, _EXPERTS), lambda i: (0, 0)),
        ],
        out_specs=pl.BlockSpec((_MM_BLOCK, _EXPERTS), lambda i: (i, 0)),
        out_shape=jax.ShapeDtypeStruct((_TOKENS, _EXPERTS), jnp.float32),
    )(x, w, b.reshape(1, _EXPERTS))


def _topk_body(logits_hbm, scores_hbm, idx_hbm, lv, sv, iv):
    wid = lax.axis_index("s") * _NC + lax.axis_index("c")
    base = wid * (_ROWS_PER_W * _EXPERTS)
    pltpu.sync_copy(logits_hbm.at[pl.ds(base, _ROWS_PER_W * _EXPERTS)], lv)

    lane = lax.iota(jnp.int32, 16)
    lo8 = lane < 8
    iconsts = [lane + 16 * j for j in range(4)]

    def merge(av, ai, bv, bi):
        rbv = lax.rev(bv, (0,))
        rbi = lax.rev(bi, (0,))
        m = av >= rbv
        mv = jnp.where(m, av, rbv)
        mi = jnp.where(m, ai, rbi)
        return plsc.sort_key_val(mv, mi, descending=True)

    def row(r, _):
        off = r * _EXPERTS
        vs = [lv[pl.ds(off + 16 * j, 16)] for j in range(4)]
        p = [
            plsc.sort_key_val(vs[j], iconsts[j], descending=True)
            for j in range(4)
        ]
        av, ai = merge(p[0][0], p[0][1], p[1][0], p[1][1])
        bv, bi = merge(p[2][0], p[2][1], p[3][0], p[3][1])
        fv, fi = merge(av, ai, bv, bi)
        mx = jnp.max(fv)
        e = jnp.where(lo8, jnp.exp(fv - mx), 0.0)
        s = jnp.sum(e)
        sc = e / s
        plsc.store_compressed(sv.at[pl.ds(r * _K, 16)], sc, mask=lo8)
        plsc.store_compressed(iv.at[pl.ds(r * _K, 16)], fi, mask=lo8)
        return ()

    lax.fori_loop(0, _ROWS_PER_W, row, (), unroll=2)

    obase = wid * (_ROWS_PER_W * _K)
    pltpu.sync_copy(
        sv.at[pl.ds(0, _ROWS_PER_W * _K)],
        scores_hbm.at[pl.ds(obase, _ROWS_PER_W * _K)],
    )
    pltpu.sync_copy(
        iv.at[pl.ds(0, _ROWS_PER_W * _K)],
        idx_hbm.at[pl.ds(obase, _ROWS_PER_W * _K)],
    )


@jax.jit
def _topk_softmax(logits_flat):
    mesh = plsc.VectorSubcoreMesh(
        core_axis_name="c", subcore_axis_name="s", num_cores=_NC,
        num_subcores=_NS,
    )
    f = functools.partial(
        pl.kernel,
        out_type=(
            jax.ShapeDtypeStruct((_TOKENS * _K,), jnp.float32),
            jax.ShapeDtypeStruct((_TOKENS * _K,), jnp.int32),
        ),
        mesh=mesh,
        compiler_params=pltpu.CompilerParams(needs_layout_passes=False),
        scratch_types=[
            pltpu.VMEM((_ROWS_PER_W * _EXPERTS,), jnp.float32),
            pltpu.VMEM((_ROWS_PER_W * _K + 8,), jnp.float32),
            pltpu.VMEM((_ROWS_PER_W * _K + 8,), jnp.int32),
        ],
    )(_topk_body)
    return f(logits_flat)


def kernel(hidden_states, weight, bias):
    x = hidden_states.reshape(-1, _HIDDEN)
    logits = _router_logits(x, weight, bias)
    scores, idx = _topk_softmax(logits.reshape(-1))
    return scores.reshape(_TOKENS, _K), idx.reshape(_TOKENS, _K)


# speedup vs baseline: 1.6111x; 1.6111x over previous
import functools

import jax
import jax.numpy as jnp
from jax import lax
from jax.experimental import pallas as pl
from jax.experimental.pallas import tpu as pltpu
from jax.experimental.pallas import tpu_sc as plsc

_TOKENS = 32768
_HIDDEN = 768
_E = 64
_K = 8
_NC = 2
_NS = 16
_NW = _NC * _NS
_RPW = _TOKENS // _NW
_MM_BLOCK = 2048

_CE19 = [(0, 1), (2, 3), (4, 5), (6, 7), (0, 2), (1, 3), (4, 6), (5, 7),
         (1, 2), (5, 6), (0, 4), (3, 7), (1, 5), (2, 6), (1, 4), (3, 6),
         (2, 4), (3, 5), (3, 4)]


def _keys_body(x_ref, w_ref, b_ref, out_ref):
    x = x_ref[...]
    w = w_ref[...]
    lg = lax.dot_general(
        w, x, (((1,), (1,)), ((), ())), preferred_element_type=jnp.float32
    )
    lg = lg + b_ref[...]
    bits = lax.bitcast_convert_type(lg, jnp.int32)
    flip = lax.shift_right_arithmetic(bits, 31) | jnp.int32(-0x80000000)
    t = lax.bitcast_convert_type(bits ^ flip, jnp.uint32)
    eid = lax.broadcasted_iota(jnp.uint32, lg.shape, 0)
    out_ref[...] = (t & jnp.uint32(0xFFFFFFC0)) | eid


def _router_keys(x, w, b):
    grid = (_TOKENS // _MM_BLOCK,)
    return pl.pallas_call(
        _keys_body,
        grid=grid,
        in_specs=[
            pl.BlockSpec((_MM_BLOCK, _HIDDEN), lambda i: (i, 0)),
            pl.BlockSpec((_E, _HIDDEN), lambda i: (0, 0)),
            pl.BlockSpec((_E, 1), lambda i: (0, 0)),
        ],
        out_specs=pl.BlockSpec((_E, _MM_BLOCK), lambda i: (0, i)),
        out_shape=jax.ShapeDtypeStruct((_E, _TOKENS), jnp.uint32),
    )(x, w, b.reshape(_E, 1))


def _ce(x, i, j):
    hi = jnp.maximum(x[i], x[j])
    lo = jnp.minimum(x[i], x[j])
    x[i] = hi
    x[j] = lo


def _sort8(x):
    for i, j in _CE19:
        _ce(x, i, j)
    return x


def _merge8(a, b):
    c = [jnp.maximum(a[i], b[7 - i]) for i in range(8)]
    for d in (4, 2, 1):
        for i in range(8):
            if (i % (2 * d)) < d:
                _ce(c, i, i + d)
    return c


def _topk_body(keys_hbm, scores_hbm, idx_hbm, kv, sv, iv):
    wid = lax.axis_index("s") * _NC + lax.axis_index("c")
    tok0 = wid * _RPW
    pltpu.sync_copy(keys_hbm.at[:, pl.ds(tok0, _RPW)], kv)

    lane = lax.iota(jnp.int32, 16)
    lane8 = lane * _K

    def group(g, _):
        col = g * 16

        def blk(b):
            return _sort8([kv[8 * b + e, pl.ds(col, 16)] for e in range(8)])

        mA = _merge8(_merge8(blk(0), blk(1)), _merge8(blk(2), blk(3)))
        mB = _merge8(_merge8(blk(4), blk(5)), _merge8(blk(6), blk(7)))
        top = _merge8(mA, mB)

        eids, vals = [], []
        for r in range(8):
            e = top[r] & jnp.uint32(63)
            t = plsc.bitcast(top[r] ^ e, jnp.int32)
            flip = (lax.shift_right_arithmetic(t, 31) ^ jnp.int32(-1)) | (
                jnp.int32(-0x80000000)
            )
            eids.append(plsc.bitcast(e, jnp.int32))
            vals.append(plsc.bitcast(t ^ flip, jnp.float32))
        exps = [jnp.exp(v - vals[0]) for v in vals]
        s = exps[0]
        for r in range(1, 8):
            s = s + exps[r]
        inv = 1.0 / s
        base = lane8 + g * (16 * _K)
        for r in range(8):
            addr = base + r
            plsc.store_scatter(sv, [addr], exps[r] * inv)
            plsc.store_scatter(iv, [addr], eids[r])
        return ()

    lax.fori_loop(0, _RPW // 16, group, (), unroll=1)

    obase = wid * (_RPW * _K)
    pltpu.sync_copy(sv, scores_hbm.at[pl.ds(obase, _RPW * _K)])
    pltpu.sync_copy(iv, idx_hbm.at[pl.ds(obase, _RPW * _K)])


def _topk_softmax(keys):
    mesh = plsc.VectorSubcoreMesh(
        core_axis_name="c", subcore_axis_name="s", num_cores=_NC,
        num_subcores=_NS,
    )
    f = functools.partial(
        pl.kernel,
        out_type=(
            jax.ShapeDtypeStruct((_TOKENS * _K,), jnp.float32),
            jax.ShapeDtypeStruct((_TOKENS * _K,), jnp.int32),
        ),
        mesh=mesh,
        compiler_params=pltpu.CompilerParams(needs_layout_passes=False),
        scratch_types=[
            pltpu.VMEM((_E, _RPW), jnp.uint32),
            pltpu.VMEM((_RPW * _K,), jnp.float32),
            pltpu.VMEM((_RPW * _K,), jnp.int32),
        ],
    )(_topk_body)
    return f(keys)


def kernel(hidden_states, weight, bias):
    x = hidden_states.reshape(-1, _HIDDEN)
    keys = _router_keys(x, weight, bias)
    scores, idx = _topk_softmax(keys)
    return scores.reshape(_TOKENS, _K), idx.reshape(_TOKENS, _K)
